# baseline (device time: 973419 ns/iter reference)
import jax
import jax.numpy as jnp
from jax import lax
from jax.experimental import pallas as pl
from jax.experimental.pallas import tpu as pltpu

N_DEV = 4
M = 4096
KS = 1024
N = 8192
NT = 256
GRID = N // NT
KSL = 256
MH = M // 2
KH = KS // 2


def kernel(x, w_mat, scale_x, scale_w):
    xb = x.astype(jnp.bfloat16)
    wt = w_mat.reshape(KS, GRID, NT).transpose(1, 0, 2)

    def body(x_hbm, w_hbm, sx_ref, sw_ref, out_ref, wg,
             xgb, wv, xseed_sem, wseed_sem, wv_sems,
             sxr, rxr, sxl, rxl, swr, rwr, swl, rwl):
        j = pl.program_id(0)
        i = pl.program_id(1)
        my = lax.axis_index("i")
        left = lax.rem(my + (N_DEV - 1), N_DEV)
        right = lax.rem(my + 1, N_DEV)

        def wv_copy(t, slot):
            return pltpu.make_async_copy(
                wg.at[:, t], wv.at[slot], wv_sems.at[slot])

        @pl.when((j == 0) & (i == 0))
        def _comm():
            cx0 = pltpu.make_async_copy(
                x_hbm.at[0:MH, :], xgb.at[my, 0], xseed_sem.at[0])
            cx1 = pltpu.make_async_copy(
                x_hbm.at[MH:M, :], xgb.at[my, 1], xseed_sem.at[0])
            cw = pltpu.make_async_copy(w_hbm, wg.at[my], wseed_sem.at[0])
            cx0.start()
            cx1.start()
            cw.start()

            barrier_sem = pltpu.get_barrier_semaphore()
            for nbr in (left, right):
                pl.semaphore_signal(barrier_sem, inc=1, device_id=(nbr,),
                                    device_id_type=pl.DeviceIdType.MESH)
            pl.semaphore_wait(barrier_sem, 2)
            cx0.wait()
            cx1.wait()
            cw.wait()

            for h in range(N_DEV - 1):
                o_r = lax.rem(my - h + N_DEV, N_DEV)
                o_l = lax.rem(my + h, N_DEV)
                rdmas = [
                    pltpu.make_async_remote_copy(
                        src_ref=xgb.at[o_r, 0], dst_ref=xgb.at[o_r, 0],
                        send_sem=sxr.at[h], recv_sem=rxr.at[h],
                        device_id=(right,),
                        device_id_type=pl.DeviceIdType.MESH),
                    pltpu.make_async_remote_copy(
                        src_ref=xgb.at[o_l, 1], dst_ref=xgb.at[o_l, 1],
                        send_sem=sxl.at[h], recv_sem=rxl.at[h],
                        device_id=(left,),
                        device_id_type=pl.DeviceIdType.MESH),
                    pltpu.make_async_remote_copy(
                        src_ref=wg.at[o_r, 0:GRID // 2],
                        dst_ref=wg.at[o_r, 0:GRID // 2],
                        send_sem=swr.at[h], recv_sem=rwr.at[h],
                        device_id=(right,),
                        device_id_type=pl.DeviceIdType.MESH),
                    pltpu.make_async_remote_copy(
                        src_ref=wg.at[o_l, GRID // 2:GRID],
                        dst_ref=wg.at[o_l, GRID // 2:GRID],
                        send_sem=swl.at[h], recv_sem=rwl.at[h],
                        device_id=(left,),
                        device_id_type=pl.DeviceIdType.MESH),
                ]
                for r in rdmas:
                    r.start()
                for r in rdmas:
                    r.wait()

            wv_copy(0, 0).start()

        slot = lax.rem(j, 2)
        nslot = lax.rem(j + 1, 2)

        @pl.when(i == 0)
        def _stream():
            wv_copy(j, slot).wait()

            @pl.when(j + 1 < GRID)
            def _prefetch():
                wv_copy(j + 1, nslot).start()

        scale = sx_ref[0] * sw_ref[0]
        for o in range(N_DEV):
            for ks in range(0, KS, KSL):
                xo = xgb[o, i, :, ks:ks + KSL]
                wo = wv[slot, o, ks:ks + KSL, :].astype(jnp.bfloat16)
                d = jnp.dot(xo, wo, preferred_element_type=jnp.float32)
                if o == 0 and ks == 0:
                    out_ref[...] = d
                else:
                    out_ref[...] += d
        out_ref[...] *= scale

    return pl.pallas_call(
        body,
        grid=(GRID, 2),
        in_specs=[
            pl.BlockSpec(memory_space=pl.ANY),
            pl.BlockSpec(memory_space=pl.ANY),
            pl.BlockSpec(memory_space=pltpu.MemorySpace.SMEM),
            pl.BlockSpec(memory_space=pltpu.MemorySpace.SMEM),
        ],
        out_specs=[
            pl.BlockSpec((MH, NT), lambda j, i: (i, j)),
            pl.BlockSpec(memory_space=pl.ANY),
        ],
        out_shape=[
            jax.ShapeDtypeStruct((M, N), jnp.float32),
            jax.ShapeDtypeStruct((N_DEV, GRID, KS, NT), jnp.int8),
        ],
        scratch_shapes=[
            pltpu.MemorySpace.VMEM((N_DEV, 2, MH, KS), jnp.bfloat16),
            pltpu.MemorySpace.VMEM((2, N_DEV, KS, NT), jnp.int8),
            pltpu.SemaphoreType.DMA((1,)),
            pltpu.SemaphoreType.DMA((1,)),
            pltpu.SemaphoreType.DMA((2,)),
            pltpu.SemaphoreType.DMA((N_DEV - 1,)),
            pltpu.SemaphoreType.DMA((N_DEV - 1,)),
            pltpu.SemaphoreType.DMA((N_DEV - 1,)),
            pltpu.SemaphoreType.DMA((N_DEV - 1,)),
            pltpu.SemaphoreType.DMA((N_DEV - 1,)),
            pltpu.SemaphoreType.DMA((N_DEV - 1,)),
            pltpu.SemaphoreType.DMA((N_DEV - 1,)),
            pltpu.SemaphoreType.DMA((N_DEV - 1,)),
        ],
        compiler_params=pltpu.CompilerParams(
            dimension_semantics=("arbitrary", "arbitrary"),
            collective_id=0,
            vmem_limit_bytes=64 * 1024 * 1024,
        ),
    )(xb, wt, scale_x, scale_w)[0]


# device time: 845514 ns/iter; 1.1513x vs baseline; 1.1513x over previous
import jax
import jax.numpy as jnp
from jax import lax
from jax.experimental import pallas as pl
from jax.experimental.pallas import tpu as pltpu

N_DEV = 4
M = 4096
KS = 1024
N = 8192
NT = 512
GRID = N // NT
KSL = 256
MH = M // 2
KH = KS // 2


def kernel(x, w_mat, scale_x, scale_w):
    def body(x_hbm, w_hbm, sx_ref, sw_ref, out_ref, wg,
             xg, wv, xseed_sem, wseed_sem, wv_sems,
             sxr, rxr, sxl, rxl, swr, rwr, swl, rwl):
        j = pl.program_id(0)
        i = pl.program_id(1)
        my = lax.axis_index("i")
        left = lax.rem(my + (N_DEV - 1), N_DEV)
        right = lax.rem(my + 1, N_DEV)

        def wv_copy(t, slot):
            return pltpu.make_async_copy(
                wg.at[:, :, pl.ds(t * NT, NT)], wv.at[slot], wv_sems.at[slot])

        @pl.when((j == 0) & (i == 0))
        def _comm():
            cx0 = pltpu.make_async_copy(
                x_hbm.at[0:MH, :], xg.at[my, 0], xseed_sem.at[0])
            cx1 = pltpu.make_async_copy(
                x_hbm.at[MH:M, :], xg.at[my, 1], xseed_sem.at[0])
            cw = pltpu.make_async_copy(w_hbm, wg.at[my], wseed_sem.at[0])
            cx0.start()
            cx1.start()
            cw.start()

            barrier_sem = pltpu.get_barrier_semaphore()
            for nbr in (left, right):
                pl.semaphore_signal(barrier_sem, inc=1, device_id=(nbr,),
                                    device_id_type=pl.DeviceIdType.MESH)
            pl.semaphore_wait(barrier_sem, 2)
            cx0.wait()
            cx1.wait()
            cw.wait()

            for h in range(N_DEV - 1):
                o_r = lax.rem(my - h + N_DEV, N_DEV)
                o_l = lax.rem(my + h, N_DEV)
                rdmas = [
                    pltpu.make_async_remote_copy(
                        src_ref=xg.at[o_r, 0], dst_ref=xg.at[o_r, 0],
                        send_sem=sxr.at[h], recv_sem=rxr.at[h],
                        device_id=(right,),
                        device_id_type=pl.DeviceIdType.MESH),
                    pltpu.make_async_remote_copy(
                        src_ref=xg.at[o_l, 1], dst_ref=xg.at[o_l, 1],
                        send_sem=sxl.at[h], recv_sem=rxl.at[h],
                        device_id=(left,),
                        device_id_type=pl.DeviceIdType.MESH),
                    pltpu.make_async_remote_copy(
                        src_ref=wg.at[o_r, 0:KH, :], dst_ref=wg.at[o_r, 0:KH, :],
                        send_sem=swr.at[h], recv_sem=rwr.at[h],
                        device_id=(right,),
                        device_id_type=pl.DeviceIdType.MESH),
                    pltpu.make_async_remote_copy(
                        src_ref=wg.at[o_l, KH:KS, :], dst_ref=wg.at[o_l, KH:KS, :],
                        send_sem=swl.at[h], recv_sem=rwl.at[h],
                        device_id=(left,),
                        device_id_type=pl.DeviceIdType.MESH),
                ]
                for r in rdmas:
                    r.start()
                for r in rdmas:
                    r.wait()

            wv_copy(0, 0).start()

        slot = lax.rem(j, 2)
        nslot = lax.rem(j + 1, 2)

        @pl.when(i == 0)
        def _stream():
            wv_copy(j, slot).wait()

            @pl.when(j + 1 < GRID)
            def _prefetch():
                wv_copy(j + 1, nslot).start()

        scale = sx_ref[0] * sw_ref[0]
        for o in range(N_DEV):
            for ks in range(0, KS, KSL):
                xo = xg[o, i, :, ks:ks + KSL].astype(jnp.bfloat16)
                wo = wv[slot, o, ks:ks + KSL, :].astype(jnp.bfloat16)
                d = jnp.dot(xo, wo, preferred_element_type=jnp.float32)
                if o == 0 and ks == 0:
                    out_ref[...] = d
                else:
                    out_ref[...] += d
        out_ref[...] *= scale

    return pl.pallas_call(
        body,
        grid=(GRID, 2),
        in_specs=[
            pl.BlockSpec(memory_space=pl.ANY),
            pl.BlockSpec(memory_space=pl.ANY),
            pl.BlockSpec(memory_space=pltpu.MemorySpace.SMEM),
            pl.BlockSpec(memory_space=pltpu.MemorySpace.SMEM),
        ],
        out_specs=[
            pl.BlockSpec((MH, NT), lambda j, i: (i, j)),
            pl.BlockSpec(memory_space=pl.ANY),
        ],
        out_shape=[
            jax.ShapeDtypeStruct((M, N), jnp.float32),
            jax.ShapeDtypeStruct((N_DEV, KS, N), jnp.int8),
        ],
        scratch_shapes=[
            pltpu.MemorySpace.VMEM((N_DEV, 2, MH, KS), jnp.int8),
            pltpu.MemorySpace.VMEM((2, N_DEV, KS, NT), jnp.int8),
            pltpu.SemaphoreType.DMA((1,)),
            pltpu.SemaphoreType.DMA((1,)),
            pltpu.SemaphoreType.DMA((2,)),
            pltpu.SemaphoreType.DMA((N_DEV - 1,)),
            pltpu.SemaphoreType.DMA((N_DEV - 1,)),
            pltpu.SemaphoreType.DMA((N_DEV - 1,)),
            pltpu.SemaphoreType.DMA((N_DEV - 1,)),
            pltpu.SemaphoreType.DMA((N_DEV - 1,)),
            pltpu.SemaphoreType.DMA((N_DEV - 1,)),
            pltpu.SemaphoreType.DMA((N_DEV - 1,)),
            pltpu.SemaphoreType.DMA((N_DEV - 1,)),
        ],
        compiler_params=pltpu.CompilerParams(
            dimension_semantics=("arbitrary", "arbitrary"),
            collective_id=0,
            vmem_limit_bytes=64 * 1024 * 1024,
        ),
    )(x, w_mat, scale_x, scale_w)[0]


# device time: 537131 ns/iter; 1.8123x vs baseline; 1.5741x over previous
import jax
import jax.numpy as jnp
from jax import lax
from jax.experimental import pallas as pl
from jax.experimental.pallas import tpu as pltpu

N_DEV = 4
M = 4096
KS = 1024
N = 8192
NH2 = N // 2
R = 128
GRID = M // R
NSL = 1024
HG = GRID // 2
KH = KS // 2


def kernel(x, w_mat, scale_x, scale_w):
    def body(x_hbm, w_hbm, sx_ref, sw_ref, out_ref,
             wgv, xgv, xseed_sem, wseed_sem,
             sxr, rxr, sxl, rxl,
             swr0, rwr0, swl0, rwl0,
             swr1, rwr1, swl1, rwl1):
        nh = pl.program_id(0)
        r = pl.program_id(1)
        my = lax.axis_index("i")
        left = lax.rem(my + (N_DEV - 1), N_DEV)
        right = lax.rem(my + 1, N_DEV)

        def w1_rdmas(h):
            o_r = lax.rem(my - h + N_DEV, N_DEV)
            o_l = lax.rem(my + h, N_DEV)
            return [
                pltpu.make_async_remote_copy(
                    src_ref=wgv.at[o_r, 1, 0:KH, :],
                    dst_ref=wgv.at[o_r, 1, 0:KH, :],
                    send_sem=swr1.at[h], recv_sem=rwr1.at[h],
                    device_id=(right,),
                    device_id_type=pl.DeviceIdType.MESH),
                pltpu.make_async_remote_copy(
                    src_ref=wgv.at[o_l, 1, KH:KS, :],
                    dst_ref=wgv.at[o_l, 1, KH:KS, :],
                    send_sem=swl1.at[h], recv_sem=rwl1.at[h],
                    device_id=(left,),
                    device_id_type=pl.DeviceIdType.MESH),
            ]

        @pl.when((nh == 0) & (r == 0))
        def _comm():
            xcopies = []
            for t in range(GRID):
                c = pltpu.make_async_copy(
                    x_hbm.at[pl.ds(t * R, R), :], xgv.at[my, t],
                    xseed_sem.at[0])
                c.start()
                xcopies.append(c)
            cw0 = pltpu.make_async_copy(
                w_hbm.at[:, 0:NH2], wgv.at[my, 0], wseed_sem.at[0])
            cw1 = pltpu.make_async_copy(
                w_hbm.at[:, NH2:N], wgv.at[my, 1], wseed_sem.at[0])
            cw0.start()
            cw1.start()

            barrier_sem = pltpu.get_barrier_semaphore()
            for nbr in (left, right):
                pl.semaphore_signal(barrier_sem, inc=1, device_id=(nbr,),
                                    device_id_type=pl.DeviceIdType.MESH)
            pl.semaphore_wait(barrier_sem, 2)
            for c in xcopies:
                c.wait()
            cw0.wait()
            cw1.wait()

            for h in range(N_DEV - 1):
                o_r = lax.rem(my - h + N_DEV, N_DEV)
                o_l = lax.rem(my + h, N_DEV)
                rdmas = [
                    pltpu.make_async_remote_copy(
                        src_ref=xgv.at[o_r, 0:HG], dst_ref=xgv.at[o_r, 0:HG],
                        send_sem=sxr.at[h], recv_sem=rxr.at[h],
                        device_id=(right,),
                        device_id_type=pl.DeviceIdType.MESH),
                    pltpu.make_async_remote_copy(
                        src_ref=xgv.at[o_l, HG:GRID],
                        dst_ref=xgv.at[o_l, HG:GRID],
                        send_sem=sxl.at[h], recv_sem=rxl.at[h],
                        device_id=(left,),
                        device_id_type=pl.DeviceIdType.MESH),
                    pltpu.make_async_remote_copy(
                        src_ref=wgv.at[o_r, 0, 0:KH, :],
                        dst_ref=wgv.at[o_r, 0, 0:KH, :],
                        send_sem=swr0.at[h], recv_sem=rwr0.at[h],
                        device_id=(right,),
                        device_id_type=pl.DeviceIdType.MESH),
                    pltpu.make_async_remote_copy(
                        src_ref=wgv.at[o_l, 0, KH:KS, :],
                        dst_ref=wgv.at[o_l, 0, KH:KS, :],
                        send_sem=swl0.at[h], recv_sem=rwl0.at[h],
                        device_id=(left,),
                        device_id_type=pl.DeviceIdType.MESH),
                ]
                for rd in rdmas:
                    rd.start()
                for rd in rdmas:
                    rd.wait()

            for rd in w1_rdmas(0):
                rd.start()

        for h in range(N_DEV - 2):
            @pl.when((nh == 0) & (r == 8 * (h + 1)))
            def _forward(h=h):
                for rd in w1_rdmas(h):
                    rd.wait()
                for rd in w1_rdmas(h + 1):
                    rd.start()

        @pl.when((nh == 1) & (r == 0))
        def _finish_w1():
            for rd in w1_rdmas(N_DEV - 2):
                rd.wait()

        scale = sx_ref[0] * sw_ref[0]
        for nn in range(0, NH2, NSL):
            acc = None
            for o in range(N_DEV):
                xo = xgv[o, r].astype(jnp.bfloat16)
                wo = wgv[o, nh, :, nn:nn + NSL].astype(jnp.bfloat16)
                d = jnp.dot(xo, wo, preferred_element_type=jnp.float32)
                acc = d if acc is None else acc + d
            out_ref[:, nn:nn + NSL] = acc * scale

    return pl.pallas_call(
        body,
        grid=(2, GRID),
        in_specs=[
            pl.BlockSpec(memory_space=pl.ANY),
            pl.BlockSpec(memory_space=pl.ANY),
            pl.BlockSpec(memory_space=pltpu.MemorySpace.SMEM),
            pl.BlockSpec(memory_space=pltpu.MemorySpace.SMEM),
        ],
        out_specs=pl.BlockSpec((R, NH2), lambda nh, r: (r, nh)),
        out_shape=jax.ShapeDtypeStruct((M, N), jnp.float32),
        scratch_shapes=[
            pltpu.MemorySpace.VMEM((N_DEV, 2, KS, NH2), jnp.int8),
            pltpu.MemorySpace.VMEM((N_DEV, GRID, R, KS), jnp.int8),
            pltpu.SemaphoreType.DMA((1,)),
            pltpu.SemaphoreType.DMA((1,)),
            pltpu.SemaphoreType.DMA((N_DEV - 1,)),
            pltpu.SemaphoreType.DMA((N_DEV - 1,)),
            pltpu.SemaphoreType.DMA((N_DEV - 1,)),
            pltpu.SemaphoreType.DMA((N_DEV - 1,)),
            pltpu.SemaphoreType.DMA((N_DEV - 1,)),
            pltpu.SemaphoreType.DMA((N_DEV - 1,)),
            pltpu.SemaphoreType.DMA((N_DEV - 1,)),
            pltpu.SemaphoreType.DMA((N_DEV - 1,)),
            pltpu.SemaphoreType.DMA((N_DEV - 1,)),
            pltpu.SemaphoreType.DMA((N_DEV - 1,)),
            pltpu.SemaphoreType.DMA((N_DEV - 1,)),
            pltpu.SemaphoreType.DMA((N_DEV - 1,)),
        ],
        compiler_params=pltpu.CompilerParams(
            dimension_semantics=("arbitrary", "arbitrary"),
            collective_id=0,
            vmem_limit_bytes=64 * 1024 * 1024,
        ),
    )(x, w_mat, scale_x, scale_w)
